# baseline (speedup 1.0000x reference)
import functools

import jax
import jax.numpy as jnp
from jax import lax
from jax.experimental import pallas as pl
from jax.experimental.pallas import tpu as pltpu
from jax.experimental.pallas import tpu_sc as plsc

_B, _F, _T = 128, 128, 1024
_NSC = 32
_NW = 32
_SPW = _NSC // _NW
_CH = 16
_NCH = _F // _CH
_NBUF = 7
_LANES = 16
_BB = 16


def _sc_body(band_hbm, x_hbm, o_hbm, band_v,
             b0, b1, b2, b3, b4, b5, b6, gsems, ssems):
    cid = lax.axis_index("c")
    sid = lax.axis_index("s")
    wid = cid * 16 + sid
    bufs = (b0, b1, b2, b3, b4, b5, b6)

    pltpu.sync_copy(band_hbm, band_v.at[pl.ds(0, _B)])
    band_vec = band_v[pl.ds(wid * _SPW, _LANES)]
    lo_vec = jnp.bitwise_and(band_vec, 0xFFFF)
    hi_vec = jnp.right_shift(band_vec, 16)

    zvec = jnp.zeros((_LANES,), jnp.float32)

    def gather(k):
        b = wid * _SPW + k // _NCH
        c = k % _NCH
        slot = k % _NBUF
        return pltpu.make_async_copy(
            x_hbm.at[b, pl.ds(c * _CH, _CH)], bufs[slot], gsems.at[slot]
        )

    def scatter(k):
        b = wid * _SPW + k // _NCH
        c = k % _NCH
        slot = k % _NBUF
        return pltpu.make_async_copy(
            bufs[slot], o_hbm.at[b, pl.ds(c * _CH, _CH)], ssems.at[slot]
        )

    nk = _SPW * _NCH
    for k in range(min(2, nk)):
        gather(k).start()

    lo = hi = None
    for k in range(nk):
        c = k % _NCH
        if c == 0:
            lo = lo_vec[k // _NCH]
            hi = hi_vec[k // _NCH]
        gather(k).wait()
        c0 = c * _CH
        s = jnp.clip(lo, c0, c0 + _CH) - c0
        e = jnp.clip(hi, c0, c0 + _CH) - c0
        buf = bufs[k % _NBUF]

        def zero_row(r, _, buf=buf):
            for seg in range(_T // _LANES):
                buf[r, pl.ds(seg * _LANES, _LANES)] = zvec
            return 0

        lax.fori_loop(s, e, zero_row, 0)
        scatter(k).start()
        if k >= 4:
            scatter(k - 4).wait()
        if k + 2 < nk:
            gather(k + 2).start()
    for k in range(max(nk - 4, 0), nk):
        scatter(k).wait()


def _tc_kernel(lo_ref, hi_ref, x_ref, o_ref, *, bb, F, T, off):
    i = pl.program_id(0)
    rows = lax.broadcasted_iota(jnp.int32, (F, T), 0)
    for j in range(bb):
        lo = lo_ref[off + i * bb + j]
        hi = hi_ref[off + i * bb + j]
        band = (rows >= lo) & (rows < hi)
        o_ref[j] = jnp.where(band, jnp.float32(0.0), x_ref[j])


def kernel(x):
    mask_ratio = 16
    xs = jnp.squeeze(x, axis=1)
    B, F, T = xs.shape
    max_mask = F // mask_ratio
    with jax.ensure_compile_time_eval():
        k = jax.random.key(42)
        k1, k2 = jax.random.split(k)
        if max_mask == 1:
            f_widths = jnp.ones((B,), dtype=jnp.int32)
        else:
            f_widths = jax.random.randint(k1, (B,), 1, max_mask).astype(jnp.int32)
        u = jax.random.uniform(k2, (B,))
        f_low = jnp.floor(u * (F - f_widths).astype(jnp.float32)).astype(jnp.int32)
        f_hi = f_low + f_widths
        band = f_low | (f_hi << 16)

    sc_run = functools.partial(
        pl.kernel,
        out_type=jax.ShapeDtypeStruct((_NSC, F, T), jnp.float32),
        mesh=plsc.VectorSubcoreMesh(core_axis_name="c", subcore_axis_name="s"),
        compiler_params=pltpu.CompilerParams(use_tc_tiling_on_sc=True),
        scratch_types=[
            pltpu.VMEM((_B + _LANES,), jnp.int32),
            pltpu.VMEM((_CH, _T), jnp.float32),
            pltpu.VMEM((_CH, _T), jnp.float32),
            pltpu.VMEM((_CH, _T), jnp.float32),
            pltpu.VMEM((_CH, _T), jnp.float32),
            pltpu.VMEM((_CH, _T), jnp.float32),
            pltpu.VMEM((_CH, _T), jnp.float32),
            pltpu.VMEM((_CH, _T), jnp.float32),
            pltpu.SemaphoreType.DMA((_NBUF,)),
            pltpu.SemaphoreType.DMA((_NBUF,)),
        ],
    )(_sc_body)
    sc_out = sc_run(band, xs)

    ntc = B - _NSC
    tc_out = pl.pallas_call(
        functools.partial(_tc_kernel, bb=_BB, F=F, T=T, off=_NSC),
        grid_spec=pltpu.PrefetchScalarGridSpec(
            num_scalar_prefetch=2,
            grid=(ntc // _BB,),
            in_specs=[
                pl.BlockSpec((_BB, F, T),
                             lambda i, lo, hi: (i + _NSC // _BB, 0, 0)),
            ],
            out_specs=pl.BlockSpec((_BB, F, T), lambda i, lo, hi: (i, 0, 0)),
        ),
        out_shape=jax.ShapeDtypeStruct((ntc, F, T), jnp.float32),
    )(f_low, f_hi, xs)

    out = jnp.concatenate([sc_out, tc_out], axis=0)
    return out[:, None, :, :]


# speedup vs baseline: 2.4405x; 2.4405x over previous
import functools

import jax
import jax.numpy as jnp
from jax import lax
from jax.experimental import pallas as pl
from jax.experimental.pallas import tpu as pltpu

_BB = 16


def _mask_kernel(lo_ref, hi_ref, x_ref, o_ref, *, bb, F, T):
    i = pl.program_id(0)
    rows = lax.broadcasted_iota(jnp.int32, (F, T), 0)
    for j in range(bb):
        lo = lo_ref[i * bb + j]
        hi = hi_ref[i * bb + j]
        band = (rows >= lo) & (rows < hi)
        o_ref[j] = jnp.where(band, jnp.float32(0.0), x_ref[j])


def kernel(x):
    mask_ratio = 16
    xs = jnp.squeeze(x, axis=1)
    B, F, T = xs.shape
    max_mask = F // mask_ratio
    with jax.ensure_compile_time_eval():
        k = jax.random.key(42)
        k1, k2 = jax.random.split(k)
        if max_mask == 1:
            f_widths = jnp.ones((B,), dtype=jnp.int32)
        else:
            f_widths = jax.random.randint(k1, (B,), 1, max_mask).astype(jnp.int32)
        u = jax.random.uniform(k2, (B,))
        f_low = jnp.floor(u * (F - f_widths).astype(jnp.float32)).astype(jnp.int32)
        f_hi = f_low + f_widths

    bb = _BB
    grid = (B // bb,)
    out = pl.pallas_call(
        functools.partial(_mask_kernel, bb=bb, F=F, T=T),
        grid_spec=pltpu.PrefetchScalarGridSpec(
            num_scalar_prefetch=2,
            grid=grid,
            in_specs=[
                pl.BlockSpec((bb, F, T), lambda i, lo, hi: (i, 0, 0)),
            ],
            out_specs=pl.BlockSpec((bb, F, T), lambda i, lo, hi: (i, 0, 0)),
        ),
        out_shape=jax.ShapeDtypeStruct((B, F, T), jnp.float32),
    )(f_low, f_hi, xs)
    return out[:, None, :, :]
